# baseline (speedup 1.0000x reference)
import functools

import jax
import jax.numpy as jnp
from jax import lax
from jax.experimental import pallas as pl
from jax.experimental.pallas import tpu as pltpu
from jax.experimental.pallas import tpu_sc as plsc

NV = 10000
NF = 100000
B = 16
NVP = 10240
E = 6 * NF

NW = 32
CHUNK = 128
NCHUNK = 147
EPT = NCHUNK * CHUNK
E_PAD = NW * EPT
PAD_COL = NV
SELF_COL = NV + 1

BM = 400
BK = 2048
N_BM = NV // BM
N_BK = NVP // BK


def _scatter_body(rows_hbm, cols_hbm, table_hbm, r_v, c_v, idx_v, ones_v, sem):
    wid = lax.axis_index("s") * 2 + lax.axis_index("c")
    base = wid * EPT
    pltpu.sync_copy(rows_hbm.at[pl.ds(base, EPT)], r_v)
    pltpu.sync_copy(cols_hbm.at[pl.ds(base, EPT)], c_v)

    @pl.loop(0, NCHUNK)
    def _compute(j):
        for t in range(CHUNK // 16):
            off = j * CHUNK + t * 16
            r = r_v[pl.ds(off, 16)]
            c = c_v[pl.ds(off, 16)]
            c = jnp.where(r == c, jnp.full((16,), SELF_COL, jnp.int32), c)
            idx_v[pl.ds(off, 16)] = r * NVP + c
            ones_v[pl.ds(off, 16)] = jnp.ones((16,), jnp.float32)

    pltpu.make_async_copy(ones_v, table_hbm.at[idx_v], sem).start()
    pltpu.make_async_copy(ones_v, table_hbm.at[idx_v], sem).wait()


@functools.cache
def _get_scatter_kernel():
    return pl.kernel(
        _scatter_body,
        out_type=(),
        mesh=plsc.VectorSubcoreMesh(core_axis_name="c", subcore_axis_name="s",
                                    num_cores=2, num_subcores=16),
        scratch_types=[
            pltpu.VMEM((EPT,), jnp.int32),
            pltpu.VMEM((EPT,), jnp.int32),
            pltpu.VMEM((EPT,), jnp.int32),
            pltpu.VMEM((EPT,), jnp.float32),
            pltpu.SemaphoreType.DMA,
        ],
    )


def _tc_body(a_ref, xe_ref, xm_ref, out_ref, acc_ref):
    m = pl.program_id(0)
    k = pl.program_id(1)

    @pl.when(k == 0)
    def _():
        acc_ref[...] = jnp.zeros_like(acc_ref)

    acc_ref[...] += jnp.dot(a_ref[...], xe_ref[...],
                            preferred_element_type=jnp.float32)

    @pl.when(k == N_BK - 1)
    def _():
        acc = acc_ref[...]
        deg = acc[:, 48:49]
        out = xm_ref[...] - acc / deg
        p = jnp.reshape(jnp.sum(out * out), (1, 1))

        @pl.when(m == 0)
        def _():
            out_ref[...] = p

        @pl.when(m > 0)
        def _():
            out_ref[...] += p

        @pl.when(m == N_BM - 1)
        def _():
            out_ref[...] = (out_ref[...] - float(NV)) / float(B)


_tc_kernel = pl.pallas_call(
    _tc_body,
    out_shape=jax.ShapeDtypeStruct((1, 1), jnp.float32),
    grid=(N_BM, N_BK),
    in_specs=[
        pl.BlockSpec((BM, BK), lambda m, k: (m, k)),
        pl.BlockSpec((BK, 64), lambda m, k: (k, 0)),
        pl.BlockSpec((BM, 64), lambda m, k: (m, 0)),
    ],
    out_specs=pl.BlockSpec((1, 1), lambda m, k: (0, 0)),
    scratch_shapes=[pltpu.VMEM((BM, 64), jnp.float32)],
)


def kernel(x, faces):
    f0 = faces[:, 0]
    f1 = faces[:, 1]
    f2 = faces[:, 2]
    rows = jnp.concatenate([f0, f1, f1, f2, f2, f0])
    cols = jnp.concatenate([f1, f0, f2, f1, f0, f2])
    pad = E_PAD - E
    rows_p = jnp.concatenate([rows, jnp.zeros((pad,), jnp.int32)])
    cols_p = jnp.concatenate([cols, jnp.full((pad,), PAD_COL, jnp.int32)])

    table_ref = jax.new_ref(jnp.zeros((NV * NVP,), jnp.float32))
    _get_scatter_kernel()(rows_p, cols_p, table_ref)
    a = table_ref[...].reshape(NV, NVP)

    xt = x.transpose(1, 0, 2).reshape(NV, B * 3)
    xe = jnp.zeros((NVP, 64), jnp.float32)
    xe = xe.at[:NV, :48].set(xt)
    xe = xe.at[:NV, 48].set(1.0)
    xe = xe.at[SELF_COL, 48].set(1.0)
    xm = jnp.zeros((NV, 64), jnp.float32).at[:, :48].set(xt)

    loss = _tc_kernel(a, xe, xm)
    return loss[0, 0]


# speedup vs baseline: 1.3503x; 1.3503x over previous
import functools

import jax
import jax.numpy as jnp
from jax import lax
from jax.experimental import pallas as pl
from jax.experimental.pallas import tpu as pltpu
from jax.experimental.pallas import tpu_sc as plsc

NV = 10000
NF = 100000
B = 16
NVP = 10240
E = 6 * NF

NW = 32
CHUNK = 128
NCHUNK = 147
EPT = NCHUNK * CHUNK
E_PAD = NW * EPT
PAD_COL = NV
SELF_COL = NV + 1

BM = 200
N_BM = NV // BM


def _scatter_body(rows_hbm, cols_hbm, table_hbm, r_v, c_v, idx_v, ones_v, sem):
    wid = lax.axis_index("s") * 2 + lax.axis_index("c")
    base = wid * EPT
    pltpu.sync_copy(rows_hbm.at[pl.ds(base, EPT)], r_v)
    pltpu.sync_copy(cols_hbm.at[pl.ds(base, EPT)], c_v)

    @pl.loop(0, NCHUNK)
    def _compute(j):
        for t in range(CHUNK // 16):
            off = j * CHUNK + t * 16
            r = r_v[pl.ds(off, 16)]
            c = c_v[pl.ds(off, 16)]
            c = jnp.where(r == c, jnp.full((16,), SELF_COL, jnp.int32), c)
            idx_v[pl.ds(off, 16)] = r * NVP + c
            ones_v[pl.ds(off, 16)] = jnp.ones((16,), jnp.float32)

    pltpu.make_async_copy(ones_v, table_hbm.at[idx_v], sem).start()
    pltpu.make_async_copy(ones_v, table_hbm.at[idx_v], sem).wait()


@functools.cache
def _get_scatter_kernel():
    return pl.kernel(
        _scatter_body,
        out_type=(),
        mesh=plsc.VectorSubcoreMesh(core_axis_name="c", subcore_axis_name="s",
                                    num_cores=2, num_subcores=16),
        scratch_types=[
            pltpu.VMEM((EPT,), jnp.int32),
            pltpu.VMEM((EPT,), jnp.int32),
            pltpu.VMEM((EPT,), jnp.int32),
            pltpu.VMEM((EPT,), jnp.float32),
            pltpu.SemaphoreType.DMA,
        ],
    )


def _tc_body(a_ref, xe_ref, xm_ref, out_ref):
    m = pl.program_id(0)
    a = a_ref[...].reshape(BM, NVP)
    acc = jnp.dot(a, xe_ref[...], preferred_element_type=jnp.float32)
    deg = acc[:, 48:49]
    out = xm_ref[...] - acc / deg
    p = jnp.reshape(jnp.sum(out * out), (1, 1))

    @pl.when(m == 0)
    def _():
        out_ref[...] = p

    @pl.when(m > 0)
    def _():
        out_ref[...] += p

    @pl.when(m == N_BM - 1)
    def _():
        out_ref[...] = (out_ref[...] - float(NV)) / float(B)


_tc_kernel = pl.pallas_call(
    _tc_body,
    out_shape=jax.ShapeDtypeStruct((1, 1), jnp.float32),
    grid=(N_BM,),
    in_specs=[
        pl.BlockSpec((BM * NVP,), lambda m: (m,)),
        pl.BlockSpec((NVP, 64), lambda m: (0, 0)),
        pl.BlockSpec((BM, 64), lambda m: (m, 0)),
    ],
    out_specs=pl.BlockSpec((1, 1), lambda m: (0, 0)),
)


def kernel(x, faces):
    f0 = faces[:, 0]
    f1 = faces[:, 1]
    f2 = faces[:, 2]
    rows = jnp.concatenate([f0, f1, f1, f2, f2, f0])
    cols = jnp.concatenate([f1, f0, f2, f1, f0, f2])
    pad = E_PAD - E
    rows_p = jnp.concatenate([rows, jnp.zeros((pad,), jnp.int32)])
    cols_p = jnp.concatenate([cols, jnp.full((pad,), PAD_COL, jnp.int32)])

    table_ref = jax.new_ref(jnp.zeros((NV * NVP,), jnp.float32))
    _get_scatter_kernel()(rows_p, cols_p, table_ref)
    a_flat = table_ref[...]

    xt = x.transpose(1, 0, 2).reshape(NV, B * 3)
    xe = jnp.zeros((NVP, 64), jnp.float32)
    xe = xe.at[:NV, :48].set(xt)
    xe = xe.at[:NV, 48].set(1.0)
    xe = xe.at[SELF_COL, 48].set(1.0)
    xm = jnp.zeros((NV, 64), jnp.float32).at[:, :48].set(xt)

    loss = _tc_kernel(a_flat, xe, xm)
    return loss[0, 0]


# speedup vs baseline: 1.3809x; 1.0226x over previous
import functools

import jax
import jax.numpy as jnp
from jax import lax
from jax.experimental import pallas as pl
from jax.experimental.pallas import tpu as pltpu
from jax.experimental.pallas import tpu_sc as plsc

NV = 10000
NF = 100000
B = 16
NVP = 10240
E2 = 3 * NF

NW = 32
CHUNK = 128
NCHUNK = 74
EPT = NCHUNK * CHUNK
E_PAD = NW * EPT
PAD_COL = NV
SELF_COL = NV + 1

BM = 200
N_BM = NV // BM
FBM = 400
N_FBM = NV // FBM


def _scatter_body(ra_hbm, rb_hbm, table_hbm, a_v, b_v, idx_v, ones_v, sem):
    wid = lax.axis_index("s") * 2 + lax.axis_index("c")
    base = wid * EPT
    pltpu.sync_copy(ra_hbm.at[pl.ds(base, EPT)], a_v)
    pltpu.sync_copy(rb_hbm.at[pl.ds(base, EPT)], b_v)

    @pl.loop(0, EPT // 16)
    def _ones(j):
        ones_v[pl.ds(j * 16, 16)] = jnp.ones((16,), jnp.float32)

    @pl.loop(0, NCHUNK)
    def _compute(j):
        for t in range(CHUNK // 16):
            off = j * CHUNK + t * 16
            a = a_v[pl.ds(off, 16)]
            b = b_v[pl.ds(off, 16)]
            r = jnp.minimum(a, b)
            c = jnp.maximum(a, b)
            c = jnp.where(a == b, jnp.full((16,), SELF_COL, jnp.int32), c)
            idx_v[pl.ds(off, 16)] = r * NVP + c

    pltpu.make_async_copy(ones_v, table_hbm.at[idx_v], sem).start()
    pltpu.make_async_copy(ones_v, table_hbm.at[idx_v], sem).wait()


@functools.cache
def _get_scatter_kernel():
    return pl.kernel(
        _scatter_body,
        out_type=(),
        mesh=plsc.VectorSubcoreMesh(core_axis_name="c", subcore_axis_name="s",
                                    num_cores=2, num_subcores=16),
        scratch_types=[
            pltpu.VMEM((EPT,), jnp.int32),
            pltpu.VMEM((EPT,), jnp.int32),
            pltpu.VMEM((EPT,), jnp.int32),
            pltpu.VMEM((EPT,), jnp.float32),
            pltpu.SemaphoreType.DMA,
        ],
    )


def _stripe_body(u_ref, xe_ref, xes_ref, out1_ref, tt_ref):
    m = pl.program_id(0)
    u = u_ref[...].reshape(BM, NVP)
    out1_ref[...] = jnp.dot(u, xe_ref[...], preferred_element_type=jnp.float32)

    @pl.when(m == 0)
    def _():
        tt_ref[...] = jnp.zeros_like(tt_ref)

    tt_ref[...] += lax.dot_general(u, xes_ref[...],
                                   (((0,), (0,)), ((), ())),
                                   preferred_element_type=jnp.float32)


_stripe_kernel = pl.pallas_call(
    _stripe_body,
    out_shape=(jax.ShapeDtypeStruct((NV, 64), jnp.float32),
               jax.ShapeDtypeStruct((NVP, 64), jnp.float32)),
    grid=(N_BM,),
    in_specs=[
        pl.BlockSpec((BM * NVP,), lambda m: (m,)),
        pl.BlockSpec((NVP, 64), lambda m: (0, 0)),
        pl.BlockSpec((BM, 64), lambda m: (m, 0)),
    ],
    out_specs=(pl.BlockSpec((BM, 64), lambda m: (m, 0)),
               pl.BlockSpec((NVP, 64), lambda m: (0, 0))),
)


def _reduce_body(o1_ref, tt_ref, xm_ref, out_ref):
    m = pl.program_id(0)
    s = o1_ref[...] + tt_ref[...]
    deg = s[:, 48:49]
    out = xm_ref[...] - s / deg
    p = jnp.reshape(jnp.sum(out * out), (1, 1))

    @pl.when(m == 0)
    def _():
        out_ref[...] = p

    @pl.when(m > 0)
    def _():
        out_ref[...] += p

    @pl.when(m == N_FBM - 1)
    def _():
        out_ref[...] = (out_ref[...] - float(NV)) / float(B)


_reduce_kernel = pl.pallas_call(
    _reduce_body,
    out_shape=jax.ShapeDtypeStruct((1, 1), jnp.float32),
    grid=(N_FBM,),
    in_specs=[
        pl.BlockSpec((FBM, 64), lambda m: (m, 0)),
        pl.BlockSpec((FBM, 64), lambda m: (m, 0)),
        pl.BlockSpec((FBM, 64), lambda m: (m, 0)),
    ],
    out_specs=pl.BlockSpec((1, 1), lambda m: (0, 0)),
)


def kernel(x, faces):
    f0 = faces[:, 0]
    f1 = faces[:, 1]
    f2 = faces[:, 2]
    ra = jnp.concatenate([f0, f1, f2])
    rb = jnp.concatenate([f1, f2, f0])
    pad = E_PAD - E2
    ra_p = jnp.concatenate([ra, jnp.zeros((pad,), jnp.int32)])
    rb_p = jnp.concatenate([rb, jnp.full((pad,), PAD_COL, jnp.int32)])

    table_ref = jax.new_ref(jnp.zeros((NV * NVP,), jnp.float32))
    _get_scatter_kernel()(ra_p, rb_p, table_ref)
    u_flat = table_ref[...]

    xt = x.transpose(1, 0, 2).reshape(NV, B * 3)
    xe = jnp.zeros((NVP, 64), jnp.float32)
    xe = xe.at[:NV, :48].set(xt)
    xe = xe.at[:NV, 48].set(1.0)
    xe = xe.at[SELF_COL, 48].set(1.0)
    xm = jnp.zeros((NV, 64), jnp.float32).at[:, :48].set(xt)

    out1, tt = _stripe_kernel(u_flat, xe, xe)
    loss = _reduce_kernel(out1, tt, xm)
    return loss[0, 0]
